# baseline (speedup 1.0000x reference)
import functools

import jax
import jax.numpy as jnp
from jax import lax
from jax.experimental import pallas as pl
from jax.experimental.pallas import tpu as pltpu
from jax.experimental.pallas import tpu_sc as plsc

TB = 512
SB = 4096
_IDX_BITS = 9
_IDX_MASK = (1 << _IDX_BITS) - 1


def _round_bf16(x):
    u = lax.bitcast_convert_type(x, jnp.uint32)
    r = (u + jnp.uint32(0x7FFF) + ((u >> 16) & jnp.uint32(1))) \
        & jnp.uint32(0xFFFF0000)
    return lax.bitcast_convert_type(r, jnp.float32)


def _select_body(a_ref, b_ref, s_ref, out_ref, mn_ref, *, ns):
    j = pl.program_id(1)

    @pl.when(j == 0)
    def _():
        mn_ref[...] = jnp.full((TB, 128), jnp.inf, dtype=jnp.float32)

    dotn = lax.dot_general(a_ref[...], b_ref[...],
                           dimension_numbers=(((0,), (0,)), ((), ())),
                           preferred_element_type=jnp.float32)
    v = s_ref[0:1, :] + dotn
    mn = mn_ref[...]
    for k in range(SB // 128):
        vk = v[:, k * 128:(k + 1) * 128]
        u = lax.bitcast_convert_type(vk, jnp.int32)
        u = (u & jnp.int32(~_IDX_MASK)) | (j * (SB // 128) + k)
        mn = jnp.minimum(mn, lax.bitcast_convert_type(u, jnp.float32))
    mn_ref[...] = mn

    @pl.when(j == ns - 1)
    def _():
        mnv = mn_ref[...]
        rm = jnp.min(mnv, axis=1)
        lane_iota = lax.broadcasted_iota(jnp.int32, (TB, 128), 1)
        lane = jnp.min(jnp.where(mnv == rm[:, None], lane_iota, 1 << 20),
                       axis=1)
        jk = lax.bitcast_convert_type(rm, jnp.int32) & _IDX_MASK
        out_ref[...] = (jk * 128 + lane).reshape(TB, 1)


def _reduce_body(g_ref, t_ref, out_ref, sum_ref, *, ng):
    i = pl.program_id(0)

    @pl.when(i == 0)
    def _():
        sum_ref[0] = 0.0

    dx = g_ref[:, 0:1] - t_ref[:, 0:1]
    dy = g_ref[:, 1:2] - t_ref[:, 1:2]
    dz = g_ref[:, 2:3] - t_ref[:, 2:3]
    d2 = dx * dx + dy * dy + dz * dz
    sum_ref[0] += jnp.sum(d2)

    @pl.when(i == ng - 1)
    def _():
        out_ref[...] = jnp.full((1, 1), sum_ref[0], jnp.float32)


def _make_sc_gather(v, d, b):
    info = plsc.get_sparse_core_info()
    nw = info.num_cores * info.num_subcores
    b_per_w = b // nw
    chunk = 512
    mesh = plsc.VectorSubcoreMesh(core_axis_name="c", subcore_axis_name="s")

    @functools.partial(
        pl.kernel, mesh=mesh,
        out_type=jax.ShapeDtypeStruct((b, d), jnp.float32),
        scratch_types=[
            pltpu.VMEM((chunk,), jnp.int32),
            pltpu.VMEM((chunk, d), jnp.float32),
            pltpu.SemaphoreType.DMA,
        ],
    )
    def k(table_hbm, idx_hbm, out_hbm, idx_v, rows_v, sem):
        wid = lax.axis_index("s") * info.num_cores + lax.axis_index("c")
        base = wid * b_per_w
        for c in range(b_per_w // chunk):
            off = base + c * chunk
            pltpu.sync_copy(idx_hbm.at[pl.ds(off, chunk)], idx_v)
            pltpu.async_copy(table_hbm.at[idx_v], rows_v, sem).wait()
            pltpu.sync_copy(rows_v, out_hbm.at[pl.ds(off, chunk)])

    return k


def kernel(src_V, tar_V):
    src_V = src_V.astype(jnp.float32)
    tar_V = tar_V.astype(jnp.float32)
    n = tar_V.shape[0]
    m = src_V.shape[0]

    tar_b = _round_bf16(tar_V)
    src_b = _round_bf16(src_V)
    src_sq = jnp.sum(src_V * src_V, axis=1, keepdims=True)

    a = jnp.concatenate(
        [tar_b, jnp.zeros((n, 13), jnp.float32)], axis=1).T.astype(jnp.bfloat16)
    bmat = jnp.concatenate(
        [-2.0 * src_b, jnp.zeros((m, 13), jnp.float32)],
        axis=1).T.astype(jnp.bfloat16)
    s = jnp.concatenate([src_sq, jnp.zeros((m, 7), jnp.float32)], axis=1).T

    nt = n // TB
    ns = m // SB

    ii = pl.pallas_call(
        functools.partial(_select_body, ns=ns),
        grid=(nt, ns),
        in_specs=[
            pl.BlockSpec((16, TB), lambda i, j: (0, i)),
            pl.BlockSpec((16, SB), lambda i, j: (0, j)),
            pl.BlockSpec((8, SB), lambda i, j: (0, j)),
        ],
        out_specs=pl.BlockSpec((TB, 1), lambda i, j: (i, 0)),
        out_shape=jax.ShapeDtypeStruct((n, 1), jnp.int32),
        scratch_shapes=[pltpu.VMEM((TB, 128), jnp.float32)],
        compiler_params=pltpu.CompilerParams(
            dimension_semantics=("parallel", "arbitrary")),
    )(a, bmat, s)

    table = jnp.concatenate([src_V, jnp.zeros((m, 125), jnp.float32)], axis=1)
    gathered = _make_sc_gather(m, 128, n)(table, ii.reshape(n))

    ng = 16
    res = pl.pallas_call(
        functools.partial(_reduce_body, ng=ng),
        grid=(ng,),
        in_specs=[
            pl.BlockSpec((n // ng, 128), lambda i: (i, 0)),
            pl.BlockSpec((n // ng, 3), lambda i: (i, 0)),
        ],
        out_specs=pl.BlockSpec((1, 1), lambda i: (0, 0)),
        out_shape=jax.ShapeDtypeStruct((1, 1), jnp.float32),
        scratch_shapes=[pltpu.SMEM((1,), jnp.float32)],
        compiler_params=pltpu.CompilerParams(
            dimension_semantics=("arbitrary",)),
    )(gathered, tar_V)
    return 0.5 * res[0, 0]


# speedup vs baseline: 1.1084x; 1.1084x over previous
import functools

import jax
import jax.numpy as jnp
from jax import lax
from jax.experimental import pallas as pl
from jax.experimental.pallas import tpu as pltpu
from jax.experimental.pallas import tpu_sc as plsc

TB = 1024
SB = 4096
_IDX_BITS = 9
_IDX_MASK = (1 << _IDX_BITS) - 1


def _round_bf16(x):
    u = lax.bitcast_convert_type(x, jnp.uint32)
    r = (u + jnp.uint32(0x7FFF) + ((u >> 16) & jnp.uint32(1))) \
        & jnp.uint32(0xFFFF0000)
    return lax.bitcast_convert_type(r, jnp.float32)


def _select_body(a_ref, b_ref, s_ref, out_ref, mn_ref, *, ns):
    j = pl.program_id(1)

    @pl.when(j == 0)
    def _():
        mn_ref[...] = jnp.full((TB, 128), jnp.inf, dtype=jnp.float32)

    dotn = lax.dot_general(a_ref[...], b_ref[...],
                           dimension_numbers=(((0,), (0,)), ((), ())),
                           preferred_element_type=jnp.float32)
    v = s_ref[0:1, :] + dotn
    mn = mn_ref[...]
    for k in range(SB // 128):
        vk = v[:, k * 128:(k + 1) * 128]
        u = lax.bitcast_convert_type(vk, jnp.int32)
        u = (u & jnp.int32(~_IDX_MASK)) | (j * (SB // 128) + k)
        mn = jnp.minimum(mn, lax.bitcast_convert_type(u, jnp.float32))
    mn_ref[...] = mn

    @pl.when(j == ns - 1)
    def _():
        mnv = mn_ref[...]
        rm = jnp.min(mnv, axis=1)
        lane_iota = lax.broadcasted_iota(jnp.int32, (TB, 128), 1)
        lane = jnp.min(jnp.where(mnv == rm[:, None], lane_iota, 1 << 20),
                       axis=1)
        jk = lax.bitcast_convert_type(rm, jnp.int32) & _IDX_MASK
        out_ref[...] = (jk * 128 + lane).reshape(TB, 1)


def _reduce_body(g_ref, t_ref, out_ref, sum_ref, *, ng):
    i = pl.program_id(0)

    @pl.when(i == 0)
    def _():
        sum_ref[0] = 0.0

    dx = g_ref[:, 0:1] - t_ref[:, 0:1]
    dy = g_ref[:, 1:2] - t_ref[:, 1:2]
    dz = g_ref[:, 2:3] - t_ref[:, 2:3]
    d2 = dx * dx + dy * dy + dz * dz
    sum_ref[0] += jnp.sum(d2)

    @pl.when(i == ng - 1)
    def _():
        out_ref[...] = jnp.full((1, 1), sum_ref[0], jnp.float32)


def _make_sc_gather(v, d, b):
    info = plsc.get_sparse_core_info()
    nw = info.num_cores * info.num_subcores
    b_per_w = b // nw
    chunk = 512
    mesh = plsc.VectorSubcoreMesh(core_axis_name="c", subcore_axis_name="s")

    @functools.partial(
        pl.kernel, mesh=mesh,
        out_type=jax.ShapeDtypeStruct((b, d), jnp.float32),
        scratch_types=[
            pltpu.VMEM((chunk,), jnp.int32),
            pltpu.VMEM((chunk, d), jnp.float32),
            pltpu.SemaphoreType.DMA,
        ],
    )
    def k(table_hbm, idx_hbm, out_hbm, idx_v, rows_v, sem):
        wid = lax.axis_index("s") * info.num_cores + lax.axis_index("c")
        base = wid * b_per_w
        for c in range(b_per_w // chunk):
            off = base + c * chunk
            pltpu.sync_copy(idx_hbm.at[pl.ds(off, chunk)], idx_v)
            pltpu.async_copy(table_hbm.at[idx_v], rows_v, sem).wait()
            pltpu.sync_copy(rows_v, out_hbm.at[pl.ds(off, chunk)])

    return k


def kernel(src_V, tar_V):
    src_V = src_V.astype(jnp.float32)
    tar_V = tar_V.astype(jnp.float32)
    n = tar_V.shape[0]
    m = src_V.shape[0]

    tar_b = _round_bf16(tar_V)
    src_b = _round_bf16(src_V)
    src_sq = jnp.sum(src_V * src_V, axis=1, keepdims=True)

    a = jnp.concatenate(
        [tar_b, jnp.zeros((n, 13), jnp.float32)], axis=1).T.astype(jnp.bfloat16)
    bmat = jnp.concatenate(
        [-2.0 * src_b, jnp.zeros((m, 13), jnp.float32)],
        axis=1).T.astype(jnp.bfloat16)
    s = jnp.concatenate([src_sq, jnp.zeros((m, 7), jnp.float32)], axis=1).T

    nt = n // TB
    ns = m // SB

    ii = pl.pallas_call(
        functools.partial(_select_body, ns=ns),
        grid=(nt, ns),
        in_specs=[
            pl.BlockSpec((16, TB), lambda i, j: (0, i)),
            pl.BlockSpec((16, SB), lambda i, j: (0, j)),
            pl.BlockSpec((8, SB), lambda i, j: (0, j)),
        ],
        out_specs=pl.BlockSpec((TB, 1), lambda i, j: (i, 0)),
        out_shape=jax.ShapeDtypeStruct((n, 1), jnp.int32),
        scratch_shapes=[pltpu.VMEM((TB, 128), jnp.float32)],
        compiler_params=pltpu.CompilerParams(
            dimension_semantics=("parallel", "arbitrary")),
    )(a, bmat, s)

    table = jnp.concatenate([src_V, jnp.zeros((m, 125), jnp.float32)], axis=1)
    gathered = _make_sc_gather(m, 128, n)(table, ii.reshape(n))

    ng = 16
    res = pl.pallas_call(
        functools.partial(_reduce_body, ng=ng),
        grid=(ng,),
        in_specs=[
            pl.BlockSpec((n // ng, 128), lambda i: (i, 0)),
            pl.BlockSpec((n // ng, 3), lambda i: (i, 0)),
        ],
        out_specs=pl.BlockSpec((1, 1), lambda i: (0, 0)),
        out_shape=jax.ShapeDtypeStruct((1, 1), jnp.float32),
        scratch_shapes=[pltpu.SMEM((1,), jnp.float32)],
        compiler_params=pltpu.CompilerParams(
            dimension_semantics=("arbitrary",)),
    )(gathered, tar_V)
    return 0.5 * res[0, 0]


# speedup vs baseline: 1.1685x; 1.0542x over previous
import functools

import jax
import jax.numpy as jnp
from jax import lax
from jax.experimental import pallas as pl
from jax.experimental.pallas import tpu as pltpu
from jax.experimental.pallas import tpu_sc as plsc

TB = 1024
SB = 8192
_IDX_BITS = 9
_IDX_MASK = (1 << _IDX_BITS) - 1


def _round_bf16(x):
    u = lax.bitcast_convert_type(x, jnp.uint32)
    r = (u + jnp.uint32(0x7FFF) + ((u >> 16) & jnp.uint32(1))) \
        & jnp.uint32(0xFFFF0000)
    return lax.bitcast_convert_type(r, jnp.float32)


def _select_body(a_ref, b_ref, s_ref, out_ref, mn_ref, *, ns):
    j = pl.program_id(1)

    @pl.when(j == 0)
    def _():
        mn_ref[...] = jnp.full((TB, 128), jnp.inf, dtype=jnp.float32)

    dotn = lax.dot_general(a_ref[...], b_ref[...],
                           dimension_numbers=(((0,), (0,)), ((), ())),
                           preferred_element_type=jnp.float32)
    v = s_ref[0:1, :] + dotn
    mn = mn_ref[...]
    for k in range(SB // 128):
        vk = v[:, k * 128:(k + 1) * 128]
        u = lax.bitcast_convert_type(vk, jnp.int32)
        u = (u & jnp.int32(~_IDX_MASK)) | (j * (SB // 128) + k)
        mn = jnp.minimum(mn, lax.bitcast_convert_type(u, jnp.float32))
    mn_ref[...] = mn

    @pl.when(j == ns - 1)
    def _():
        mnv = mn_ref[...]
        rm = jnp.min(mnv, axis=1)
        lane_iota = lax.broadcasted_iota(jnp.int32, (TB, 128), 1)
        lane = jnp.min(jnp.where(mnv == rm[:, None], lane_iota, 1 << 20),
                       axis=1)
        jk = lax.bitcast_convert_type(rm, jnp.int32) & _IDX_MASK
        out_ref[...] = (jk * 128 + lane).reshape(TB, 1)


def _reduce_body(g_ref, t_ref, out_ref, sum_ref, *, ng):
    i = pl.program_id(0)

    @pl.when(i == 0)
    def _():
        sum_ref[0] = 0.0

    dx = g_ref[:, 0:1] - t_ref[:, 0:1]
    dy = g_ref[:, 1:2] - t_ref[:, 1:2]
    dz = g_ref[:, 2:3] - t_ref[:, 2:3]
    d2 = dx * dx + dy * dy + dz * dz
    sum_ref[0] += jnp.sum(d2)

    @pl.when(i == ng - 1)
    def _():
        out_ref[...] = jnp.full((1, 1), sum_ref[0], jnp.float32)


def _make_sc_gather(v, d, b):
    info = plsc.get_sparse_core_info()
    nw = info.num_cores * info.num_subcores
    b_per_w = b // nw
    chunk = 512
    mesh = plsc.VectorSubcoreMesh(core_axis_name="c", subcore_axis_name="s")

    @functools.partial(
        pl.kernel, mesh=mesh,
        out_type=jax.ShapeDtypeStruct((b, d), jnp.float32),
        scratch_types=[
            pltpu.VMEM((chunk,), jnp.int32),
            pltpu.VMEM((chunk, d), jnp.float32),
            pltpu.SemaphoreType.DMA,
        ],
    )
    def k(table_hbm, idx_hbm, out_hbm, idx_v, rows_v, sem):
        wid = lax.axis_index("s") * info.num_cores + lax.axis_index("c")
        base = wid * b_per_w
        for c in range(b_per_w // chunk):
            off = base + c * chunk
            pltpu.sync_copy(idx_hbm.at[pl.ds(off, chunk)], idx_v)
            pltpu.async_copy(table_hbm.at[idx_v], rows_v, sem).wait()
            pltpu.sync_copy(rows_v, out_hbm.at[pl.ds(off, chunk)])

    return k


def kernel(src_V, tar_V):
    src_V = src_V.astype(jnp.float32)
    tar_V = tar_V.astype(jnp.float32)
    n = tar_V.shape[0]
    m = src_V.shape[0]

    tar_b = _round_bf16(tar_V)
    src_b = _round_bf16(src_V)
    src_sq = jnp.sum(src_V * src_V, axis=1, keepdims=True)

    a = jnp.concatenate(
        [tar_b, jnp.zeros((n, 13), jnp.float32)], axis=1).T.astype(jnp.bfloat16)
    bmat = jnp.concatenate(
        [-2.0 * src_b, jnp.zeros((m, 13), jnp.float32)],
        axis=1).T.astype(jnp.bfloat16)
    s = jnp.concatenate([src_sq, jnp.zeros((m, 7), jnp.float32)], axis=1).T

    nt = n // TB
    ns = m // SB

    ii = pl.pallas_call(
        functools.partial(_select_body, ns=ns),
        grid=(nt, ns),
        in_specs=[
            pl.BlockSpec((16, TB), lambda i, j: (0, i)),
            pl.BlockSpec((16, SB), lambda i, j: (0, j)),
            pl.BlockSpec((8, SB), lambda i, j: (0, j)),
        ],
        out_specs=pl.BlockSpec((TB, 1), lambda i, j: (i, 0)),
        out_shape=jax.ShapeDtypeStruct((n, 1), jnp.int32),
        scratch_shapes=[pltpu.VMEM((TB, 128), jnp.float32)],
        compiler_params=pltpu.CompilerParams(
            dimension_semantics=("parallel", "arbitrary")),
    )(a, bmat, s)

    table = jnp.concatenate([src_V, jnp.zeros((m, 125), jnp.float32)], axis=1)
    gathered = _make_sc_gather(m, 128, n)(table, ii.reshape(n))

    ng = 16
    res = pl.pallas_call(
        functools.partial(_reduce_body, ng=ng),
        grid=(ng,),
        in_specs=[
            pl.BlockSpec((n // ng, 128), lambda i: (i, 0)),
            pl.BlockSpec((n // ng, 3), lambda i: (i, 0)),
        ],
        out_specs=pl.BlockSpec((1, 1), lambda i: (0, 0)),
        out_shape=jax.ShapeDtypeStruct((1, 1), jnp.float32),
        scratch_shapes=[pltpu.SMEM((1,), jnp.float32)],
        compiler_params=pltpu.CompilerParams(
            dimension_semantics=("arbitrary",)),
    )(gathered, tar_V)
    return 0.5 * res[0, 0]


# speedup vs baseline: 1.2042x; 1.0305x over previous
import functools

import jax
import jax.numpy as jnp
from jax import lax
from jax.experimental import pallas as pl
from jax.experimental.pallas import tpu as pltpu
from jax.experimental.pallas import tpu_sc as plsc

TB = 2048
SB = 8192
_IDX_BITS = 9
_IDX_MASK = (1 << _IDX_BITS) - 1


def _round_bf16(x):
    u = lax.bitcast_convert_type(x, jnp.uint32)
    r = (u + jnp.uint32(0x7FFF) + ((u >> 16) & jnp.uint32(1))) \
        & jnp.uint32(0xFFFF0000)
    return lax.bitcast_convert_type(r, jnp.float32)


def _select_body(a_ref, b_ref, s_ref, out_ref, mn_ref, *, ns):
    j = pl.program_id(1)

    @pl.when(j == 0)
    def _():
        mn_ref[...] = jnp.full((TB, 128), jnp.inf, dtype=jnp.float32)

    dotn = lax.dot_general(a_ref[...], b_ref[...],
                           dimension_numbers=(((0,), (0,)), ((), ())),
                           preferred_element_type=jnp.float32)
    v = s_ref[0:1, :] + dotn
    mn = mn_ref[...]
    for k in range(SB // 128):
        vk = v[:, k * 128:(k + 1) * 128]
        u = lax.bitcast_convert_type(vk, jnp.int32)
        u = (u & jnp.int32(~_IDX_MASK)) | (j * (SB // 128) + k)
        mn = jnp.minimum(mn, lax.bitcast_convert_type(u, jnp.float32))
    mn_ref[...] = mn

    @pl.when(j == ns - 1)
    def _():
        mnv = mn_ref[...]
        rm = jnp.min(mnv, axis=1)
        lane_iota = lax.broadcasted_iota(jnp.int32, (TB, 128), 1)
        lane = jnp.min(jnp.where(mnv == rm[:, None], lane_iota, 1 << 20),
                       axis=1)
        jk = lax.bitcast_convert_type(rm, jnp.int32) & _IDX_MASK
        out_ref[...] = (jk * 128 + lane).reshape(TB, 1)


def _reduce_body(g_ref, t_ref, out_ref, sum_ref, *, ng):
    i = pl.program_id(0)

    @pl.when(i == 0)
    def _():
        sum_ref[0] = 0.0

    dx = g_ref[:, 0:1] - t_ref[:, 0:1]
    dy = g_ref[:, 1:2] - t_ref[:, 1:2]
    dz = g_ref[:, 2:3] - t_ref[:, 2:3]
    d2 = dx * dx + dy * dy + dz * dz
    sum_ref[0] += jnp.sum(d2)

    @pl.when(i == ng - 1)
    def _():
        out_ref[...] = jnp.full((1, 1), sum_ref[0], jnp.float32)


def _make_sc_gather(v, d, b):
    info = plsc.get_sparse_core_info()
    nw = info.num_cores * info.num_subcores
    b_per_w = b // nw
    chunk = 512
    mesh = plsc.VectorSubcoreMesh(core_axis_name="c", subcore_axis_name="s")

    @functools.partial(
        pl.kernel, mesh=mesh,
        out_type=jax.ShapeDtypeStruct((b, d), jnp.float32),
        scratch_types=[
            pltpu.VMEM((chunk,), jnp.int32),
            pltpu.VMEM((chunk, d), jnp.float32),
            pltpu.SemaphoreType.DMA,
        ],
    )
    def k(table_hbm, idx_hbm, out_hbm, idx_v, rows_v, sem):
        wid = lax.axis_index("s") * info.num_cores + lax.axis_index("c")
        base = wid * b_per_w
        for c in range(b_per_w // chunk):
            off = base + c * chunk
            pltpu.sync_copy(idx_hbm.at[pl.ds(off, chunk)], idx_v)
            pltpu.async_copy(table_hbm.at[idx_v], rows_v, sem).wait()
            pltpu.sync_copy(rows_v, out_hbm.at[pl.ds(off, chunk)])

    return k


def kernel(src_V, tar_V):
    src_V = src_V.astype(jnp.float32)
    tar_V = tar_V.astype(jnp.float32)
    n = tar_V.shape[0]
    m = src_V.shape[0]

    tar_b = _round_bf16(tar_V)
    src_b = _round_bf16(src_V)
    src_sq = jnp.sum(src_V * src_V, axis=1, keepdims=True)

    a = jnp.concatenate(
        [tar_b, jnp.zeros((n, 13), jnp.float32)], axis=1).T.astype(jnp.bfloat16)
    bmat = jnp.concatenate(
        [-2.0 * src_b, jnp.zeros((m, 13), jnp.float32)],
        axis=1).T.astype(jnp.bfloat16)
    s = jnp.concatenate([src_sq, jnp.zeros((m, 7), jnp.float32)], axis=1).T

    nt = n // TB
    ns = m // SB

    ii = pl.pallas_call(
        functools.partial(_select_body, ns=ns),
        grid=(nt, ns),
        in_specs=[
            pl.BlockSpec((16, TB), lambda i, j: (0, i)),
            pl.BlockSpec((16, SB), lambda i, j: (0, j)),
            pl.BlockSpec((8, SB), lambda i, j: (0, j)),
        ],
        out_specs=pl.BlockSpec((TB, 1), lambda i, j: (i, 0)),
        out_shape=jax.ShapeDtypeStruct((n, 1), jnp.int32),
        scratch_shapes=[pltpu.VMEM((TB, 128), jnp.float32)],
        compiler_params=pltpu.CompilerParams(
            dimension_semantics=("parallel", "arbitrary")),
    )(a, bmat, s)

    table = jnp.concatenate([src_V, jnp.zeros((m, 125), jnp.float32)], axis=1)
    gathered = _make_sc_gather(m, 128, n)(table, ii.reshape(n))

    ng = 16
    res = pl.pallas_call(
        functools.partial(_reduce_body, ng=ng),
        grid=(ng,),
        in_specs=[
            pl.BlockSpec((n // ng, 128), lambda i: (i, 0)),
            pl.BlockSpec((n // ng, 3), lambda i: (i, 0)),
        ],
        out_specs=pl.BlockSpec((1, 1), lambda i: (0, 0)),
        out_shape=jax.ShapeDtypeStruct((1, 1), jnp.float32),
        scratch_shapes=[pltpu.SMEM((1,), jnp.float32)],
        compiler_params=pltpu.CompilerParams(
            dimension_semantics=("arbitrary",)),
    )(gathered, tar_V)
    return 0.5 * res[0, 0]


# speedup vs baseline: 1.2270x; 1.0190x over previous
import functools

import jax
import jax.numpy as jnp
from jax import lax
from jax.experimental import pallas as pl
from jax.experimental.pallas import tpu as pltpu
from jax.experimental.pallas import tpu_sc as plsc

TB = 4096
SB = 8192
_IDX_BITS = 9
_IDX_MASK = (1 << _IDX_BITS) - 1


def _round_bf16(x):
    u = lax.bitcast_convert_type(x, jnp.uint32)
    r = (u + jnp.uint32(0x7FFF) + ((u >> 16) & jnp.uint32(1))) \
        & jnp.uint32(0xFFFF0000)
    return lax.bitcast_convert_type(r, jnp.float32)


def _select_body(a_ref, b_ref, s_ref, out_ref, mn_ref, *, ns):
    j = pl.program_id(1)

    @pl.when(j == 0)
    def _():
        mn_ref[...] = jnp.full((TB, 128), jnp.inf, dtype=jnp.float32)

    dotn = lax.dot_general(a_ref[...], b_ref[...],
                           dimension_numbers=(((0,), (0,)), ((), ())),
                           preferred_element_type=jnp.float32)
    v = s_ref[0:1, :] + dotn
    mn = mn_ref[...]
    for k in range(SB // 128):
        vk = v[:, k * 128:(k + 1) * 128]
        u = lax.bitcast_convert_type(vk, jnp.int32)
        u = (u & jnp.int32(~_IDX_MASK)) | (j * (SB // 128) + k)
        mn = jnp.minimum(mn, lax.bitcast_convert_type(u, jnp.float32))
    mn_ref[...] = mn

    @pl.when(j == ns - 1)
    def _():
        mnv = mn_ref[...]
        rm = jnp.min(mnv, axis=1)
        lane_iota = lax.broadcasted_iota(jnp.int32, (TB, 128), 1)
        lane = jnp.min(jnp.where(mnv == rm[:, None], lane_iota, 1 << 20),
                       axis=1)
        jk = lax.bitcast_convert_type(rm, jnp.int32) & _IDX_MASK
        out_ref[...] = (jk * 128 + lane).reshape(TB, 1)


def _reduce_body(g_ref, t_ref, out_ref, sum_ref, *, ng):
    i = pl.program_id(0)

    @pl.when(i == 0)
    def _():
        sum_ref[0] = 0.0

    dx = g_ref[:, 0:1] - t_ref[:, 0:1]
    dy = g_ref[:, 1:2] - t_ref[:, 1:2]
    dz = g_ref[:, 2:3] - t_ref[:, 2:3]
    d2 = dx * dx + dy * dy + dz * dz
    sum_ref[0] += jnp.sum(d2)

    @pl.when(i == ng - 1)
    def _():
        out_ref[...] = jnp.full((1, 1), sum_ref[0], jnp.float32)


def _make_sc_gather(v, d, b):
    info = plsc.get_sparse_core_info()
    nw = info.num_cores * info.num_subcores
    b_per_w = b // nw
    chunk = 512
    mesh = plsc.VectorSubcoreMesh(core_axis_name="c", subcore_axis_name="s")

    @functools.partial(
        pl.kernel, mesh=mesh,
        out_type=jax.ShapeDtypeStruct((b, d), jnp.float32),
        scratch_types=[
            pltpu.VMEM((chunk,), jnp.int32),
            pltpu.VMEM((chunk, d), jnp.float32),
            pltpu.SemaphoreType.DMA,
        ],
    )
    def k(table_hbm, idx_hbm, out_hbm, idx_v, rows_v, sem):
        wid = lax.axis_index("s") * info.num_cores + lax.axis_index("c")
        base = wid * b_per_w
        for c in range(b_per_w // chunk):
            off = base + c * chunk
            pltpu.sync_copy(idx_hbm.at[pl.ds(off, chunk)], idx_v)
            pltpu.async_copy(table_hbm.at[idx_v], rows_v, sem).wait()
            pltpu.sync_copy(rows_v, out_hbm.at[pl.ds(off, chunk)])

    return k


def kernel(src_V, tar_V):
    src_V = src_V.astype(jnp.float32)
    tar_V = tar_V.astype(jnp.float32)
    n = tar_V.shape[0]
    m = src_V.shape[0]

    tar_b = _round_bf16(tar_V)
    src_b = _round_bf16(src_V)
    src_sq = jnp.sum(src_V * src_V, axis=1, keepdims=True)

    a = jnp.concatenate(
        [tar_b, jnp.zeros((n, 13), jnp.float32)], axis=1).T.astype(jnp.bfloat16)
    bmat = jnp.concatenate(
        [-2.0 * src_b, jnp.zeros((m, 13), jnp.float32)],
        axis=1).T.astype(jnp.bfloat16)
    s = jnp.concatenate([src_sq, jnp.zeros((m, 7), jnp.float32)], axis=1).T

    nt = n // TB
    ns = m // SB

    ii = pl.pallas_call(
        functools.partial(_select_body, ns=ns),
        grid=(nt, ns),
        in_specs=[
            pl.BlockSpec((16, TB), lambda i, j: (0, i)),
            pl.BlockSpec((16, SB), lambda i, j: (0, j)),
            pl.BlockSpec((8, SB), lambda i, j: (0, j)),
        ],
        out_specs=pl.BlockSpec((TB, 1), lambda i, j: (i, 0)),
        out_shape=jax.ShapeDtypeStruct((n, 1), jnp.int32),
        scratch_shapes=[pltpu.VMEM((TB, 128), jnp.float32)],
        compiler_params=pltpu.CompilerParams(
            dimension_semantics=("parallel", "arbitrary")),
    )(a, bmat, s)

    table = jnp.concatenate([src_V, jnp.zeros((m, 125), jnp.float32)], axis=1)
    gathered = _make_sc_gather(m, 128, n)(table, ii.reshape(n))

    ng = 16
    res = pl.pallas_call(
        functools.partial(_reduce_body, ng=ng),
        grid=(ng,),
        in_specs=[
            pl.BlockSpec((n // ng, 128), lambda i: (i, 0)),
            pl.BlockSpec((n // ng, 3), lambda i: (i, 0)),
        ],
        out_specs=pl.BlockSpec((1, 1), lambda i: (0, 0)),
        out_shape=jax.ShapeDtypeStruct((1, 1), jnp.float32),
        scratch_shapes=[pltpu.SMEM((1,), jnp.float32)],
        compiler_params=pltpu.CompilerParams(
            dimension_semantics=("arbitrary",)),
    )(gathered, tar_V)
    return 0.5 * res[0, 0]
